# baseline (speedup 1.0000x reference)
import jax
import jax.numpy as jnp
from jax.experimental import pallas as pl

_K = 1000
_KPAD = 1024
_BM = 1024


def _kmeans_block(x_ref, c_ref, dist_ref, label_ref):
    xb = x_ref[...]
    c = c_ref[...]
    a2 = jnp.sum(xb * xb, axis=1)
    idx = jax.lax.broadcasted_iota(jnp.int32, (_KPAD,), 0)
    b2 = jnp.where(idx < _K, jnp.sum(c * c, axis=1), jnp.inf)
    xc = jax.lax.dot_general(
        xb, c * -2.0, (((1,), (1,)), ((), ())),
        preferred_element_type=jnp.float32)
    t = xc + b2[None, :]
    m = jnp.min(t, axis=1)
    col = jax.lax.broadcasted_iota(jnp.int32, t.shape, 1)
    lbl = jnp.min(jnp.where(t == m[:, None], col, _KPAD), axis=1)
    label_ref[...] = lbl
    dist_ref[...] = jnp.sqrt(jnp.maximum(m + a2, 1e-12))


@jax.jit
def kernel(x, centers):
    n = x.shape[0]
    c_pad = jnp.zeros((_KPAD, centers.shape[1]), centers.dtype)
    c_pad = c_pad.at[:_K].set(centers)
    grid = (n // _BM,)
    dist, labels = pl.pallas_call(
        _kmeans_block,
        grid=grid,
        in_specs=[
            pl.BlockSpec((_BM, x.shape[1]), lambda i: (i, 0)),
            pl.BlockSpec((_KPAD, centers.shape[1]), lambda i: (0, 0)),
        ],
        out_specs=[
            pl.BlockSpec((_BM,), lambda i: (i,)),
            pl.BlockSpec((_BM,), lambda i: (i,)),
        ],
        out_shape=[
            jax.ShapeDtypeStruct((n,), jnp.float32),
            jax.ShapeDtypeStruct((n,), jnp.int32),
        ],
    )(x, c_pad)
    return dist, labels


# speedup vs baseline: 1.0145x; 1.0145x over previous
import jax
import jax.numpy as jnp
from jax.experimental import pallas as pl

_K = 1000
_KPAD = 1024
_BM = 1024


def _kmeans_block(x_ref, c_ref, dist_ref, label_ref):
    xb = x_ref[...]
    c = c_ref[...]
    a2 = jnp.sum(xb * xb, axis=1)
    idx = jax.lax.broadcasted_iota(jnp.int32, (_KPAD,), 0)
    b2 = jnp.where(idx < _K, jnp.sum(c * c, axis=1), jnp.inf)
    xc = jax.lax.dot_general(
        xb, c * -2.0, (((1,), (1,)), ((), ())),
        preferred_element_type=jnp.float32)
    t = xc + b2[None, :]
    v = t[:, 0:128]
    gi = jnp.zeros(v.shape, jnp.int32)
    for g in range(1, _KPAD // 128):
        s = t[:, g * 128:(g + 1) * 128]
        gi = jnp.where(s < v, g, gi)
        v = jnp.minimum(v, s)
    rm = jnp.min(v, axis=1)
    lane = jax.lax.broadcasted_iota(jnp.int32, v.shape, 1)
    fi = gi * 128 + lane
    lbl = jnp.min(jnp.where(v == rm[:, None], fi, 1 << 20), axis=1)
    label_ref[...] = lbl
    dist_ref[...] = jnp.sqrt(jnp.maximum(rm + a2, 1e-12))


@jax.jit
def kernel(x, centers):
    n = x.shape[0]
    c_pad = jnp.zeros((_KPAD, centers.shape[1]), centers.dtype)
    c_pad = c_pad.at[:_K].set(centers)
    grid = (n // _BM,)
    dist, labels = pl.pallas_call(
        _kmeans_block,
        grid=grid,
        in_specs=[
            pl.BlockSpec((_BM, x.shape[1]), lambda i: (i, 0)),
            pl.BlockSpec((_KPAD, centers.shape[1]), lambda i: (0, 0)),
        ],
        out_specs=[
            pl.BlockSpec((_BM,), lambda i: (i,)),
            pl.BlockSpec((_BM,), lambda i: (i,)),
        ],
        out_shape=[
            jax.ShapeDtypeStruct((n,), jnp.float32),
            jax.ShapeDtypeStruct((n,), jnp.int32),
        ],
    )(x, c_pad)
    return dist, labels


# speedup vs baseline: 2.9736x; 2.9312x over previous
import jax
import jax.numpy as jnp
from jax.experimental import pallas as pl

_K = 1000
_KPAD = 1024
_BM = 1024


def _kmeans_block(x_ref, c_ref, dist_ref, label_ref):
    xb = x_ref[...]
    c = c_ref[...]
    b2 = jnp.sum(c * c, axis=1, keepdims=True)
    kidx = jax.lax.broadcasted_iota(jnp.int32, (_KPAD, 1), 0)
    b2m = jnp.where(kidx < _K, b2, jnp.inf)
    t = jax.lax.dot_general(
        c * -2.0, xb, (((1,), (1,)), ((), ())),
        preferred_element_type=jnp.float32) + b2m
    ones8 = jnp.ones((8, xb.shape[1]), jnp.float32)
    a2 = jax.lax.dot_general(
        ones8, xb * xb, (((1,), (1,)), ((), ())),
        preferred_element_type=jnp.float32)[0]
    v = t[0:8, :]
    ri = jnp.zeros(v.shape, jnp.int32)
    for r in range(1, _KPAD // 8):
        s = t[8 * r:8 * (r + 1), :]
        ri = jnp.where(s < v, r, ri)
        v = jnp.minimum(v, s)
    si = jax.lax.broadcasted_iota(jnp.int32, v.shape, 0)
    fullidx = ri * 8 + si
    m = jnp.min(v, axis=0)
    lbl = jnp.min(jnp.where(v == m[None, :], fullidx, 1 << 20), axis=0)
    label_ref[...] = lbl
    dist_ref[...] = jnp.sqrt(jnp.maximum(m + a2, 1e-12))


@jax.jit
def kernel(x, centers):
    n = x.shape[0]
    c_pad = jnp.zeros((_KPAD, centers.shape[1]), centers.dtype)
    c_pad = c_pad.at[:_K].set(centers)
    grid = (n // _BM,)
    dist, labels = pl.pallas_call(
        _kmeans_block,
        grid=grid,
        in_specs=[
            pl.BlockSpec((_BM, x.shape[1]), lambda i: (i, 0)),
            pl.BlockSpec((_KPAD, centers.shape[1]), lambda i: (0, 0)),
        ],
        out_specs=[
            pl.BlockSpec((_BM,), lambda i: (i,)),
            pl.BlockSpec((_BM,), lambda i: (i,)),
        ],
        out_shape=[
            jax.ShapeDtypeStruct((n,), jnp.float32),
            jax.ShapeDtypeStruct((n,), jnp.int32),
        ],
    )(x, c_pad)
    return dist, labels


# speedup vs baseline: 3.5000x; 1.1770x over previous
import jax
import jax.numpy as jnp
from jax.experimental import pallas as pl

_K = 1000
_KPAD = 1024
_BM = 2048


def _kmeans_block(x_ref, c_ref, dist_ref, label_ref):
    xb = x_ref[...]
    c = c_ref[...]
    b2 = jnp.sum(c * c, axis=1, keepdims=True)
    kidx = jax.lax.broadcasted_iota(jnp.int32, (_KPAD, 1), 0)
    b2m = jnp.where(kidx < _K, b2, jnp.inf)
    t = jax.lax.dot_general(
        c * -2.0, xb, (((1,), (1,)), ((), ())),
        preferred_element_type=jnp.float32) + b2m
    ones8 = jnp.ones((8, xb.shape[1]), jnp.float32)
    a2 = jax.lax.dot_general(
        ones8, xb * xb, (((1,), (1,)), ((), ())),
        preferred_element_type=jnp.float32)[0]
    v = t[0:8, :]
    ri = jnp.zeros(v.shape, jnp.int32)
    for r in range(1, _KPAD // 8):
        s = t[8 * r:8 * (r + 1), :]
        ri = jnp.where(s < v, r, ri)
        v = jnp.minimum(v, s)
    si = jax.lax.broadcasted_iota(jnp.int32, v.shape, 0)
    fullidx = ri * 8 + si
    m = jnp.min(v, axis=0)
    lbl = jnp.min(jnp.where(v == m[None, :], fullidx, 1 << 20), axis=0)
    label_ref[...] = lbl
    dist_ref[...] = jnp.sqrt(jnp.maximum(m + a2, 1e-12))


@jax.jit
def kernel(x, centers):
    n = x.shape[0]
    c_pad = jnp.zeros((_KPAD, centers.shape[1]), centers.dtype)
    c_pad = c_pad.at[:_K].set(centers)
    grid = (n // _BM,)
    dist, labels = pl.pallas_call(
        _kmeans_block,
        grid=grid,
        in_specs=[
            pl.BlockSpec((_BM, x.shape[1]), lambda i: (i, 0)),
            pl.BlockSpec((_KPAD, centers.shape[1]), lambda i: (0, 0)),
        ],
        out_specs=[
            pl.BlockSpec((_BM,), lambda i: (i,)),
            pl.BlockSpec((_BM,), lambda i: (i,)),
        ],
        out_shape=[
            jax.ShapeDtypeStruct((n,), jnp.float32),
            jax.ShapeDtypeStruct((n,), jnp.int32),
        ],
    )(x, c_pad)
    return dist, labels


# speedup vs baseline: 3.6375x; 1.0393x over previous
import jax
import jax.numpy as jnp
from jax.experimental import pallas as pl

_K = 1000
_KPAD = 1024
_BM = 4096


def _kmeans_block(x_ref, c_ref, dist_ref, label_ref):
    xb = x_ref[...]
    c = c_ref[...]
    b2 = jnp.sum(c * c, axis=1, keepdims=True)
    kidx = jax.lax.broadcasted_iota(jnp.int32, (_KPAD, 1), 0)
    b2m = jnp.where(kidx < _K, b2, jnp.inf)
    t = jax.lax.dot_general(
        c * -2.0, xb, (((1,), (1,)), ((), ())),
        preferred_element_type=jnp.float32) + b2m
    ones8 = jnp.ones((8, xb.shape[1]), jnp.float32)
    a2 = jax.lax.dot_general(
        ones8, xb * xb, (((1,), (1,)), ((), ())),
        preferred_element_type=jnp.float32)[0]
    v = t[0:8, :]
    ri = jnp.zeros(v.shape, jnp.int32)
    for r in range(1, _KPAD // 8):
        s = t[8 * r:8 * (r + 1), :]
        ri = jnp.where(s < v, r, ri)
        v = jnp.minimum(v, s)
    si = jax.lax.broadcasted_iota(jnp.int32, v.shape, 0)
    fullidx = ri * 8 + si
    m = jnp.min(v, axis=0)
    lbl = jnp.min(jnp.where(v == m[None, :], fullidx, 1 << 20), axis=0)
    label_ref[...] = lbl
    dist_ref[...] = jnp.sqrt(jnp.maximum(m + a2, 1e-12))


@jax.jit
def kernel(x, centers):
    n = x.shape[0]
    c_pad = jnp.zeros((_KPAD, centers.shape[1]), centers.dtype)
    c_pad = c_pad.at[:_K].set(centers)
    grid = (n // _BM,)
    dist, labels = pl.pallas_call(
        _kmeans_block,
        grid=grid,
        in_specs=[
            pl.BlockSpec((_BM, x.shape[1]), lambda i: (i, 0)),
            pl.BlockSpec((_KPAD, centers.shape[1]), lambda i: (0, 0)),
        ],
        out_specs=[
            pl.BlockSpec((_BM,), lambda i: (i,)),
            pl.BlockSpec((_BM,), lambda i: (i,)),
        ],
        out_shape=[
            jax.ShapeDtypeStruct((n,), jnp.float32),
            jax.ShapeDtypeStruct((n,), jnp.int32),
        ],
    )(x, c_pad)
    return dist, labels


# speedup vs baseline: 3.6521x; 1.0040x over previous
import jax
import jax.numpy as jnp
from jax.experimental import pallas as pl

_K = 1000
_KPAD = 1024
_BM = 8192


def _kmeans_block(x_ref, c_ref, dist_ref, label_ref):
    xb = x_ref[...]
    c = c_ref[...]
    b2 = jnp.sum(c * c, axis=1, keepdims=True)
    kidx = jax.lax.broadcasted_iota(jnp.int32, (_KPAD, 1), 0)
    b2m = jnp.where(kidx < _K, b2, jnp.inf)
    t = jax.lax.dot_general(
        c * -2.0, xb, (((1,), (1,)), ((), ())),
        preferred_element_type=jnp.float32) + b2m
    ones8 = jnp.ones((8, xb.shape[1]), jnp.float32)
    a2 = jax.lax.dot_general(
        ones8, xb * xb, (((1,), (1,)), ((), ())),
        preferred_element_type=jnp.float32)[0]
    v = t[0:8, :]
    ri = jnp.zeros(v.shape, jnp.int32)
    for r in range(1, _KPAD // 8):
        s = t[8 * r:8 * (r + 1), :]
        ri = jnp.where(s < v, r, ri)
        v = jnp.minimum(v, s)
    si = jax.lax.broadcasted_iota(jnp.int32, v.shape, 0)
    fullidx = ri * 8 + si
    m = jnp.min(v, axis=0)
    lbl = jnp.min(jnp.where(v == m[None, :], fullidx, 1 << 20), axis=0)
    label_ref[...] = lbl
    dist_ref[...] = jnp.sqrt(jnp.maximum(m + a2, 1e-12))


@jax.jit
def kernel(x, centers):
    n = x.shape[0]
    c_pad = jnp.zeros((_KPAD, centers.shape[1]), centers.dtype)
    c_pad = c_pad.at[:_K].set(centers)
    grid = (n // _BM,)
    dist, labels = pl.pallas_call(
        _kmeans_block,
        grid=grid,
        in_specs=[
            pl.BlockSpec((_BM, x.shape[1]), lambda i: (i, 0)),
            pl.BlockSpec((_KPAD, centers.shape[1]), lambda i: (0, 0)),
        ],
        out_specs=[
            pl.BlockSpec((_BM,), lambda i: (i,)),
            pl.BlockSpec((_BM,), lambda i: (i,)),
        ],
        out_shape=[
            jax.ShapeDtypeStruct((n,), jnp.float32),
            jax.ShapeDtypeStruct((n,), jnp.int32),
        ],
    )(x, c_pad)
    return dist, labels


# speedup vs baseline: 4.1689x; 1.1415x over previous
import jax
import jax.numpy as jnp
from jax.experimental import pallas as pl

_K = 1000
_BM = 8192


def _kmeans_block(x_ref, c_ref, dist_ref, label_ref):
    xb = x_ref[...]
    c = c_ref[...]
    b2 = jnp.sum(c * c, axis=1, keepdims=True)
    t = jax.lax.dot_general(
        c * -2.0, xb, (((1,), (1,)), ((), ())),
        preferred_element_type=jnp.float32) + b2
    ones8 = jnp.ones((8, xb.shape[1]), jnp.float32)
    a2 = jax.lax.dot_general(
        ones8, xb * xb, (((1,), (1,)), ((), ())),
        preferred_element_type=jnp.float32)[0]
    v = t[0:8, :]
    ri = jnp.zeros(v.shape, jnp.int32)
    for r in range(1, _K // 8):
        s = t[8 * r:8 * (r + 1), :]
        ri = jnp.where(s < v, r, ri)
        v = jnp.minimum(v, s)
    si = jax.lax.broadcasted_iota(jnp.int32, v.shape, 0)
    fullidx = ri * 8 + si
    m = jnp.min(v, axis=0)
    lbl = jnp.min(jnp.where(v == m[None, :], fullidx, 1 << 20), axis=0)
    label_ref[...] = lbl
    dist_ref[...] = jnp.sqrt(jnp.maximum(m + a2, 1e-12))


@jax.jit
def kernel(x, centers):
    n = x.shape[0]
    grid = (n // _BM,)
    dist, labels = pl.pallas_call(
        _kmeans_block,
        grid=grid,
        in_specs=[
            pl.BlockSpec((_BM, x.shape[1]), lambda i: (i, 0)),
            pl.BlockSpec((_K, centers.shape[1]), lambda i: (0, 0)),
        ],
        out_specs=[
            pl.BlockSpec((_BM,), lambda i: (i,)),
            pl.BlockSpec((_BM,), lambda i: (i,)),
        ],
        out_shape=[
            jax.ShapeDtypeStruct((n,), jnp.float32),
            jax.ShapeDtypeStruct((n,), jnp.int32),
        ],
    )(x, centers)
    return dist, labels


# speedup vs baseline: 4.1898x; 1.0050x over previous
import jax
import jax.numpy as jnp
from jax.experimental import pallas as pl

_K = 1000
_BM = 4096


def _kmeans_block(x_ref, c_ref, dist_ref, label_ref):
    xb = x_ref[...]
    c = c_ref[...]
    b2 = jnp.sum(c * c, axis=1, keepdims=True)
    t = jax.lax.dot_general(
        c * -2.0, xb, (((1,), (1,)), ((), ())),
        preferred_element_type=jnp.float32) + b2
    ones8 = jnp.ones((8, xb.shape[1]), jnp.float32)
    a2 = jax.lax.dot_general(
        ones8, xb * xb, (((1,), (1,)), ((), ())),
        preferred_element_type=jnp.float32)[0]
    v = t[0:8, :]
    ri = jnp.zeros(v.shape, jnp.int32)
    for r in range(1, _K // 8):
        s = t[8 * r:8 * (r + 1), :]
        ri = jnp.where(s < v, r, ri)
        v = jnp.minimum(v, s)
    si = jax.lax.broadcasted_iota(jnp.int32, v.shape, 0)
    fullidx = ri * 8 + si
    m = jnp.min(v, axis=0)
    lbl = jnp.min(jnp.where(v == m[None, :], fullidx, 1 << 20), axis=0)
    label_ref[...] = lbl
    dist_ref[...] = jnp.sqrt(jnp.maximum(m + a2, 1e-12))


@jax.jit
def kernel(x, centers):
    n = x.shape[0]
    grid = (n // _BM,)
    dist, labels = pl.pallas_call(
        _kmeans_block,
        grid=grid,
        in_specs=[
            pl.BlockSpec((_BM, x.shape[1]), lambda i: (i, 0)),
            pl.BlockSpec((_K, centers.shape[1]), lambda i: (0, 0)),
        ],
        out_specs=[
            pl.BlockSpec((_BM,), lambda i: (i,)),
            pl.BlockSpec((_BM,), lambda i: (i,)),
        ],
        out_shape=[
            jax.ShapeDtypeStruct((n,), jnp.float32),
            jax.ShapeDtypeStruct((n,), jnp.int32),
        ],
    )(x, centers)
    return dist, labels
